# baseline (speedup 1.0000x reference)
import dataclasses
import functools

import jax
import jax.numpy as jnp
from jax import lax
from jax.experimental import pallas as pl
from jax.experimental.pallas import tpu as pltpu
from jax.experimental.pallas import tpu_sc as plsc

CUTOFF = 5.2
NMOL = 100
NATOM = 500
NELEM = 4

NC = 2
NS = 16
NW = NC * NS
LANES = 16
TILE = 128
BINS_W = 128

_FC_C = (
    0.99999972183970045,
    -0.24999779082777177,
    0.020830516395037845,
    -0.00069313736579340925,
    1.2126596247947612e-05,
    -1.1096974968826522e-07,
)


def _sc_pair_kernel(n_pairs, n_atoms_total, ct, unroll):
    n_tiles = n_pairs // TILE
    n_chunks = n_tiles // ct
    n_rounds = -(-n_chunks // NW)
    full_w = n_chunks - (n_rounds - 1) * NW
    mesh = plsc.VectorSubcoreMesh(
        core_axis_name="c", subcore_axis_name="s",
        num_cores=NC, num_subcores=NS)
    inv_cut = float(jnp.pi) / CUTOFF

    cp = pltpu.CompilerParams()
    if "needs_layout_passes" in pltpu.CompilerParams.__dataclass_fields__:
        cp = dataclasses.replace(cp, needs_layout_passes=False)

    @functools.partial(
        pl.kernel,
        mesh=mesh,
        compiler_params=cp,
        out_type=jax.ShapeDtypeStruct((NW, BINS_W), jnp.float32),
        scratch_types=[
            pltpu.VMEM((n_atoms_total,), jnp.int32),
            pltpu.VMEM((NELEM * NELEM,), jnp.float32),
            pltpu.VMEM((NELEM * NELEM,), jnp.float32),
            pltpu.VMEM((LANES, BINS_W), jnp.float32),
            pltpu.VMEM((BINS_W,), jnp.float32),
            pltpu.VMEM((ct, 2, TILE), jnp.int32),
            pltpu.VMEM((ct, 2, TILE), jnp.int32),
            pltpu.VMEM((ct * TILE,), jnp.float32),
            pltpu.VMEM((ct * TILE,), jnp.float32),
            pltpu.SemaphoreType.DMA,
            pltpu.SemaphoreType.DMA,
        ],
    )
    def k(packed_hbm, it_hbm, d_hbm, a_hbm, b_hbm, out_hbm,
          packed_v, a_v, b_v, bins_v, acc_v, iv_a, iv_b, d_a, d_b,
          sem0, sem1):
        iv_bufs = (iv_a, iv_b)
        d_bufs = (d_a, d_b)
        sems = (sem0, sem1)
        wid = lax.axis_index("s") * NC + lax.axis_index("c")
        in_last = wid < full_w

        def chunk_idx(r):
            ci = wid + r * NW
            return lax.select(ci < n_chunks, ci, wid)

        tbl = pltpu.async_copy(packed_hbm, packed_v, sem0)
        pltpu.sync_copy(a_hbm, a_v)
        pltpu.sync_copy(b_hbm, b_v)

        zeros = jnp.zeros((LANES,), jnp.float32)

        @pl.loop(0, LANES)
        def _zero_rows(r):
            @pl.loop(0, BINS_W, step=LANES)
            def _zero_cols(cc):
                bins_v[r, pl.ds(cc, LANES)] = zeros

        rows = lax.iota(jnp.int32, LANES)

        def copies(ci, slot):
            s = sems[slot]
            return [
                pltpu.make_async_copy(it_hbm.at[pl.ds(ci * ct, ct)],
                                      iv_bufs[slot], s),
                pltpu.make_async_copy(d_hbm.at[pl.ds(ci * ct * TILE,
                                                     ct * TILE)],
                                      d_bufs[slot], s),
            ]

        def start2(r, slot):
            for cp_ in copies(chunk_idx(r), slot):
                cp_.start()

        def wait2(r, slot):
            for cp_ in copies(chunk_idx(r), slot):
                cp_.wait()

        def compute(slot):
            iv = iv_bufs[slot]
            dv = d_bufs[slot]

            @plsc.parallel_loop(0, ct, 1, unroll=unroll)
            def _vec(g):
                db = g * TILE
                for t in range(TILE // LANES):
                    sl = pl.ds(t * LANES, LANES)
                    vi0 = iv[g, 0, sl]
                    vi1 = iv[g, 1, sl]
                    p0 = plsc.load_gather(packed_v, [vi0])
                    p1 = plsc.load_gather(packed_v, [vi1])
                    c = (p0 + (p1 >> 2)) & 15
                    m = p0 >> 10
                    av = plsc.load_gather(a_v, [c])
                    bv = plsc.load_gather(b_v, [c])
                    d = dv[pl.ds(db + t * LANES, LANES)]
                    inv = 1.0 / d
                    inv2 = inv * inv
                    inv6 = inv2 * inv2 * inv2
                    x = d * inv_cut
                    u = x * x
                    fc = jnp.float32(_FC_C[5])
                    for cf in (_FC_C[4], _FC_C[3], _FC_C[2],
                               _FC_C[1], _FC_C[0]):
                        fc = fc * u + jnp.float32(cf)
                    tt = inv6 * fc
                    e = tt * (av * inv6 + bv)
                    plsc.addupdate_scatter(bins_v, [rows, m], e)

        assert n_rounds >= 2 and n_rounds % 2 == 0
        start2(0, 0)
        tbl.wait()
        wait2(0, 0)
        start2(1, 1)
        compute(0)

        @pl.loop(0, (n_rounds - 2) // 2)
        def _rounds(dr):
            r1 = 1 + 2 * dr
            wait2(r1, 1)
            start2(r1 + 1, 0)
            compute(1)
            wait2(r1 + 1, 0)
            start2(r1 + 2, 1)
            compute(0)

        wait2(n_rounds - 1, 1)

        @pl.when(in_last)
        def _last():
            compute(1)

        for cc in range(0, BINS_W, LANES):
            sl = pl.ds(cc, LANES)
            s = bins_v[0, sl]
            for r in range(1, LANES):
                s = s + bins_v[r, sl]
            acc_v[sl] = s
        pltpu.sync_copy(acc_v, out_hbm.at[wid])

    return k


def _tc_pack(x_ref, o_ref):
    n_mols, n_atoms = x_ref.shape
    mol = lax.broadcasted_iota(jnp.int32, (n_mols, n_atoms), 0)
    o_ref[...] = mol * 1024 + x_ref[...] * 4


def _tc_reduce(x_ref, o_ref):
    o_ref[...] = jnp.sum(x_ref[...], axis=0, keepdims=True)


def kernel(element_idxs, indices, distances, eps, sigma):
    n_mols, n_atoms = element_idxs.shape
    n_pairs = distances.shape[0]
    packed = pl.pallas_call(
        _tc_pack,
        out_shape=jax.ShapeDtypeStruct((n_mols, n_atoms), jnp.int32),
    )(element_idxs).reshape(-1)
    sig2 = sigma * sigma
    sig6 = sig2 * sig2 * sig2
    a = (4.0 * eps * sig6 * sig6).reshape(-1)
    b = (-4.0 * eps * sig6).reshape(-1)
    it = indices.reshape(2, n_pairs // TILE, TILE).transpose(1, 0, 2)
    sc = _sc_pair_kernel(n_pairs, n_mols * n_atoms, 50, 4)
    partials = sc(packed, it, distances, a, b)
    reduced = pl.pallas_call(
        _tc_reduce,
        out_shape=jax.ShapeDtypeStruct((1, BINS_W), jnp.float32),
    )(partials)
    return reduced[0, :n_mols]


# speedup vs baseline: 1.7101x; 1.7101x over previous
import dataclasses
import functools

import jax
import jax.numpy as jnp
from jax import lax
from jax.experimental import pallas as pl
from jax.experimental.pallas import tpu as pltpu
from jax.experimental.pallas import tpu_sc as plsc

CUTOFF = 5.2
NMOL = 100
NATOM = 500
NELEM = 4

NC = 2
NS = 16
NW = NC * NS
LANES = 16
TILE = 128
BINS_W = 128

_FC_C = (
    0.9999997218397004,
    -0.09124923434232221,
    0.002775148021736263,
    -3.3705322418581135e-05,
    2.1523413563597094e-07,
    -7.189023396294283e-10,
)
_RCP_MAGIC = 0x7EF311C3


def _sc_pair_kernel(n_pairs, n_atoms_total, ct, unroll):
    n_tiles = n_pairs // TILE
    n_chunks = n_tiles // ct
    n_rounds = -(-n_chunks // NW)
    full_w = n_chunks - (n_rounds - 1) * NW
    mesh = plsc.VectorSubcoreMesh(
        core_axis_name="c", subcore_axis_name="s",
        num_cores=NC, num_subcores=NS)
    inv_cut = float(jnp.pi) / CUTOFF

    cp = pltpu.CompilerParams()
    if "needs_layout_passes" in pltpu.CompilerParams.__dataclass_fields__:
        cp = dataclasses.replace(cp, needs_layout_passes=False)

    @functools.partial(
        pl.kernel,
        mesh=mesh,
        compiler_params=cp,
        out_type=jax.ShapeDtypeStruct((NW, BINS_W), jnp.float32),
        scratch_types=[
            pltpu.VMEM((n_atoms_total,), jnp.int32),
            pltpu.VMEM((NELEM * NELEM,), jnp.float32),
            pltpu.VMEM((NELEM * NELEM,), jnp.float32),
            pltpu.VMEM((LANES, BINS_W), jnp.float32),
            pltpu.VMEM((BINS_W,), jnp.float32),
            pltpu.VMEM((ct, 2, TILE), jnp.int32),
            pltpu.VMEM((ct, 2, TILE), jnp.int32),
            pltpu.VMEM((ct * TILE,), jnp.float32),
            pltpu.VMEM((ct * TILE,), jnp.float32),
            pltpu.SemaphoreType.DMA,
            pltpu.SemaphoreType.DMA,
        ],
    )
    def k(packed_hbm, it_hbm, d_hbm, a_hbm, b_hbm, out_hbm,
          packed_v, a_v, b_v, bins_v, acc_v, iv_a, iv_b, d_a, d_b,
          sem0, sem1):
        iv_bufs = (iv_a, iv_b)
        d_bufs = (d_a, d_b)
        sems = (sem0, sem1)
        wid = lax.axis_index("s") * NC + lax.axis_index("c")
        in_last = wid < full_w

        def chunk_idx(r):
            ci = wid + r * NW
            return lax.select(ci < n_chunks, ci, wid)

        tbl = pltpu.async_copy(packed_hbm, packed_v, sem0)
        pltpu.sync_copy(a_hbm, a_v)
        pltpu.sync_copy(b_hbm, b_v)

        zeros = jnp.zeros((LANES,), jnp.float32)

        @pl.loop(0, LANES)
        def _zero_rows(r):
            @pl.loop(0, BINS_W, step=LANES)
            def _zero_cols(cc):
                bins_v[r, pl.ds(cc, LANES)] = zeros

        rows = lax.iota(jnp.int32, LANES)

        def copies(ci, slot):
            s = sems[slot]
            return [
                pltpu.make_async_copy(it_hbm.at[pl.ds(ci * ct, ct)],
                                      iv_bufs[slot], s),
                pltpu.make_async_copy(d_hbm.at[pl.ds(ci * ct * TILE,
                                                     ct * TILE)],
                                      d_bufs[slot], s),
            ]

        def start2(r, slot):
            for cp_ in copies(chunk_idx(r), slot):
                cp_.start()

        def wait2(r, slot):
            for cp_ in copies(chunk_idx(r), slot):
                cp_.wait()

        def compute(slot):
            iv = iv_bufs[slot]
            dv = d_bufs[slot]

            @plsc.parallel_loop(0, ct, 1, unroll=unroll)
            def _vec(g):
                db = g * TILE
                for t in range(TILE // LANES):
                    sl = pl.ds(t * LANES, LANES)
                    vi0 = iv[g, 0, sl]
                    vi1 = iv[g, 1, sl]
                    p0 = plsc.load_gather(packed_v, [vi0])
                    p1 = plsc.load_gather(packed_v, [vi1])
                    c = (p0 + (p1 >> 2)) & 15
                    m = p0 >> 10
                    av = plsc.load_gather(a_v, [c])
                    bv = plsc.load_gather(b_v, [c])
                    d = dv[pl.ds(db + t * LANES, LANES)]
                    inv = lax.bitcast_convert_type(
                        jnp.int32(_RCP_MAGIC)
                        - lax.bitcast_convert_type(d, jnp.int32),
                        jnp.float32)
                    inv = inv * (2.0 - d * inv)
                    inv = inv * (2.0 - d * inv)
                    inv2 = inv * inv
                    inv6 = inv2 * inv2 * inv2
                    u = d * d
                    fc = jnp.float32(_FC_C[5])
                    for cf in (_FC_C[4], _FC_C[3], _FC_C[2],
                               _FC_C[1], _FC_C[0]):
                        fc = fc * u + jnp.float32(cf)
                    tt = inv6 * fc
                    e = tt * (av * inv6 + bv)
                    plsc.addupdate_scatter(bins_v, [rows, m], e)

        assert n_rounds >= 2 and n_rounds % 2 == 0
        start2(0, 0)
        tbl.wait()
        wait2(0, 0)
        start2(1, 1)
        compute(0)

        @pl.loop(0, (n_rounds - 2) // 2)
        def _rounds(dr):
            r1 = 1 + 2 * dr
            wait2(r1, 1)
            start2(r1 + 1, 0)
            compute(1)
            wait2(r1 + 1, 0)
            start2(r1 + 2, 1)
            compute(0)

        wait2(n_rounds - 1, 1)

        @pl.when(in_last)
        def _last():
            compute(1)

        for cc in range(0, BINS_W, LANES):
            sl = pl.ds(cc, LANES)
            s = bins_v[0, sl]
            for r in range(1, LANES):
                s = s + bins_v[r, sl]
            acc_v[sl] = s
        pltpu.sync_copy(acc_v, out_hbm.at[wid])

    return k


def _tc_pack(x_ref, o_ref):
    n_mols, n_atoms = x_ref.shape
    mol = lax.broadcasted_iota(jnp.int32, (n_mols, n_atoms), 0)
    o_ref[...] = mol * 1024 + x_ref[...] * 4


def _tc_reduce(x_ref, o_ref):
    o_ref[...] = jnp.sum(x_ref[...], axis=0)[:NMOL]


def kernel(element_idxs, indices, distances, eps, sigma):
    n_mols, n_atoms = element_idxs.shape
    n_pairs = distances.shape[0]
    packed = pl.pallas_call(
        _tc_pack,
        out_shape=jax.ShapeDtypeStruct((n_mols, n_atoms), jnp.int32),
    )(element_idxs).reshape(-1)
    sig2 = sigma * sigma
    sig6 = sig2 * sig2 * sig2
    a = (4.0 * eps * sig6 * sig6).reshape(-1)
    b = (-4.0 * eps * sig6).reshape(-1)
    it = indices.reshape(2, n_pairs // TILE, TILE).transpose(1, 0, 2)
    sc = _sc_pair_kernel(n_pairs, n_mols * n_atoms, 50, 2)
    partials = sc(packed, it, distances, a, b)
    return pl.pallas_call(
        _tc_reduce,
        out_shape=jax.ShapeDtypeStruct((n_mols,), jnp.float32),
    )(partials)


# speedup vs baseline: 1.7919x; 1.0479x over previous
import dataclasses
import functools

import jax
import jax.numpy as jnp
from jax import lax
from jax.experimental import pallas as pl
from jax.experimental.pallas import tpu as pltpu
from jax.experimental.pallas import tpu_sc as plsc

CUTOFF = 5.2
NMOL = 100
NATOM = 500
NELEM = 4

NC = 2
NS = 16
NW = NC * NS
LANES = 16
TILE = 128
BINS_W = 128

_FC_C = (
    0.9999997218397004,
    -0.09124923434232221,
    0.002775148021736263,
    -3.3705322418581135e-05,
    2.1523413563597094e-07,
    -7.189023396294283e-10,
)
_RCP_MAGIC = 0x7EF311C3


def _sc_pair_kernel(n_pairs, n_atoms_total, ct, unroll):
    n_tiles = n_pairs // TILE
    n_chunks = n_tiles // ct
    n_rounds = -(-n_chunks // NW)
    full_w = n_chunks - (n_rounds - 1) * NW
    mesh = plsc.VectorSubcoreMesh(
        core_axis_name="c", subcore_axis_name="s",
        num_cores=NC, num_subcores=NS)
    inv_cut = float(jnp.pi) / CUTOFF

    cp = pltpu.CompilerParams()
    if "needs_layout_passes" in pltpu.CompilerParams.__dataclass_fields__:
        cp = dataclasses.replace(cp, needs_layout_passes=False)

    @functools.partial(
        pl.kernel,
        mesh=mesh,
        compiler_params=cp,
        out_type=jax.ShapeDtypeStruct((NW, BINS_W), jnp.float32),
        scratch_types=[
            pltpu.VMEM((n_atoms_total,), jnp.int32),
            pltpu.VMEM((NELEM * NELEM,), jnp.float32),
            pltpu.VMEM((NELEM * NELEM,), jnp.float32),
            pltpu.VMEM((LANES, BINS_W), jnp.float32),
            pltpu.VMEM((BINS_W,), jnp.float32),
            pltpu.VMEM((ct, 2, TILE), jnp.int32),
            pltpu.VMEM((ct, 2, TILE), jnp.int32),
            pltpu.VMEM((ct * TILE,), jnp.float32),
            pltpu.VMEM((ct * TILE,), jnp.float32),
            pltpu.SemaphoreType.DMA,
            pltpu.SemaphoreType.DMA,
        ],
    )
    def k(packed_hbm, it_hbm, d_hbm, a_hbm, b_hbm, out_hbm,
          packed_v, a_v, b_v, bins_v, acc_v, iv_a, iv_b, d_a, d_b,
          sem0, sem1):
        iv_bufs = (iv_a, iv_b)
        d_bufs = (d_a, d_b)
        sems = (sem0, sem1)
        wid = lax.axis_index("s") * NC + lax.axis_index("c")
        in_last = wid < full_w

        def chunk_idx(r):
            ci = wid + r * NW
            return lax.select(ci < n_chunks, ci, wid)

        tbl = pltpu.async_copy(packed_hbm, packed_v, sem0)
        pltpu.sync_copy(a_hbm, a_v)
        pltpu.sync_copy(b_hbm, b_v)

        zeros = jnp.zeros((LANES,), jnp.float32)

        @pl.loop(0, LANES)
        def _zero_rows(r):
            @pl.loop(0, BINS_W, step=LANES)
            def _zero_cols(cc):
                bins_v[r, pl.ds(cc, LANES)] = zeros

        rows = lax.iota(jnp.int32, LANES)

        def copies(ci, slot):
            s = sems[slot]
            return [
                pltpu.make_async_copy(it_hbm.at[pl.ds(ci * ct, ct)],
                                      iv_bufs[slot], s),
                pltpu.make_async_copy(d_hbm.at[pl.ds(ci * ct * TILE,
                                                     ct * TILE)],
                                      d_bufs[slot], s),
            ]

        def start2(r, slot):
            for cp_ in copies(chunk_idx(r), slot):
                cp_.start()

        def wait2(r, slot):
            for cp_ in copies(chunk_idx(r), slot):
                cp_.wait()

        def compute(slot):
            iv = iv_bufs[slot]
            dv = d_bufs[slot]

            @plsc.parallel_loop(0, ct, 1, unroll=unroll)
            def _vec(g):
                db = g * TILE
                for t in range(TILE // LANES):
                    sl = pl.ds(t * LANES, LANES)
                    vi0 = iv[g, 0, sl]
                    vi1 = iv[g, 1, sl]
                    p0 = plsc.load_gather(packed_v, [vi0])
                    p1 = plsc.load_gather(packed_v, [vi1])
                    c = (p0 + (p1 >> 2)) & 15
                    m = p0 >> 10
                    av = plsc.load_gather(a_v, [c])
                    bv = plsc.load_gather(b_v, [c])
                    d = dv[pl.ds(db + t * LANES, LANES)]
                    inv = 1.0 / d
                    inv2 = inv * inv
                    inv6 = inv2 * inv2 * inv2
                    u = d * d
                    fc = jnp.float32(_FC_C[5])
                    for cf in (_FC_C[4], _FC_C[3], _FC_C[2],
                               _FC_C[1], _FC_C[0]):
                        fc = fc * u + jnp.float32(cf)
                    tt = inv6 * fc
                    e = tt * (av * inv6 + bv)
                    plsc.addupdate_scatter(bins_v, [rows, m], e)

        assert n_rounds >= 2 and n_rounds % 2 == 0
        half = n_rounds // 2
        start2(0, 0)
        tbl.wait()

        @pl.loop(0, half)
        def _rounds(dr):
            r0 = 2 * dr
            wait2(r0, 0)
            start2(r0 + 1, 1)
            compute(0)
            wait2(r0 + 1, 1)

            @pl.when(dr + 1 < half)
            def _start_next():
                start2(r0 + 2, 0)

            @pl.when(jnp.logical_or(dr + 1 < half, in_last))
            def _compute_b():
                compute(1)

        for cc in range(0, BINS_W, LANES):
            sl = pl.ds(cc, LANES)
            s = bins_v[0, sl]
            for r in range(1, LANES):
                s = s + bins_v[r, sl]
            acc_v[sl] = s
        pltpu.sync_copy(acc_v, out_hbm.at[wid])

    return k


def _tc_pack(x_ref, o_ref):
    n_mols, n_atoms = x_ref.shape
    mol = lax.broadcasted_iota(jnp.int32, (n_mols, n_atoms), 0)
    o_ref[...] = mol * 1024 + x_ref[...] * 4


def _tc_reduce(x_ref, o_ref):
    o_ref[...] = jnp.sum(x_ref[...], axis=0)[:NMOL]


def kernel(element_idxs, indices, distances, eps, sigma):
    n_mols, n_atoms = element_idxs.shape
    n_pairs = distances.shape[0]
    packed = pl.pallas_call(
        _tc_pack,
        out_shape=jax.ShapeDtypeStruct((n_mols, n_atoms), jnp.int32),
    )(element_idxs).reshape(-1)
    sig2 = sigma * sigma
    sig6 = sig2 * sig2 * sig2
    a = (4.0 * eps * sig6 * sig6).reshape(-1)
    b = (-4.0 * eps * sig6).reshape(-1)
    it = indices.reshape(2, n_pairs // TILE, TILE).transpose(1, 0, 2)
    sc = _sc_pair_kernel(n_pairs, n_mols * n_atoms, 50, 2)
    partials = sc(packed, it, distances, a, b)
    return pl.pallas_call(
        _tc_reduce,
        out_shape=jax.ShapeDtypeStruct((n_mols,), jnp.float32),
    )(partials)
